# baseline (speedup 1.0000x reference)
import functools

import jax
import jax.numpy as jnp
from jax import lax
from jax.experimental import pallas as pl
from jax.experimental.pallas import tpu as pltpu
from jax.experimental.pallas import tpu_sc as plsc

TOTAL = 32768
D = 128
NC = 2
NS = 16
L = 16
NW = NC * NS
TPW = TOTAL // NW
CH = 64
NCH = TPW // CH
G = CH // L
NB = 4

_mesh = plsc.VectorSubcoreMesh(core_axis_name="c", subcore_axis_name="s")


@functools.partial(
    pl.kernel,
    mesh=_mesh,
    out_type=jax.ShapeDtypeStruct((TOTAL,), jnp.float32),
    compiler_params=pltpu.CompilerParams(needs_layout_passes=False),
    scratch_types=[
        pltpu.VMEM((TPW,), jnp.int32),
        pltpu.VMEM((NB, CH, D), jnp.float32),
        pltpu.VMEM((TPW,), jnp.float32),
        pltpu.VMEM((D, 1), jnp.float32),
        pltpu.VMEM((1,), jnp.float32),
        pltpu.SemaphoreType.DMA((NB,)),
    ],
)
def _sc_cost_kernel(tokens_hbm, table_hbm, w_hbm, b_hbm, out_hbm,
                    idx_v, rows, out_v, w_v, b_v, sems):
    wid = lax.axis_index("s") * NC + lax.axis_index("c")
    base = wid * TPW

    pltpu.sync_copy(w_hbm, w_v)
    pltpu.sync_copy(b_hbm, b_v)
    pltpu.sync_copy(tokens_hbm.at[pl.ds(base, TPW)], idx_v)

    def gather_copy(c):
        slot = lax.rem(c, NB)
        return pltpu.make_async_copy(
            table_hbm.at[idx_v.at[pl.ds(pl.multiple_of(c * CH, CH), CH)]],
            rows.at[slot], sems.at[slot])

    lane = lax.broadcasted_iota(jnp.int32, (L,), 0)
    zero16 = jnp.zeros((L,), jnp.int32)
    bias_vec = plsc.load_gather(b_v, [zero16])
    ridx = tuple(lane + g * L for g in range(G))

    for i in range(NB - 1):
        gather_copy(i).start()

    def chunk_body(c, _):
        @pl.when(c + NB - 1 < NCH)
        def _():
            gather_copy(c + NB - 1).start()

        slot = lax.rem(c, NB)
        gather_copy(c).wait()
        bsplat = zero16 + slot

        def body(i, accs):
            k = i & (L - 1)
            blk = i - k
            boff = ((lane + k) & (L - 1)) + blk
            wv = plsc.load_gather(w_v, [boff, zero16])
            return tuple(
                accs[g] + plsc.load_gather(rows, [bsplat, ridx[g], boff]) * wv
                for g in range(G))

        accs = lax.fori_loop(0, D, body, (bias_vec,) * G)
        for g in range(G):
            out_v[pl.ds(c * CH + g * L, L)] = accs[g]
        return 0

    lax.fori_loop(0, NCH, chunk_body, 0)
    pltpu.sync_copy(out_v, out_hbm.at[pl.ds(base, TPW)])


def kernel(tokens, cu_seqlens, table, W, b):
    return _sc_cost_kernel(tokens, table, W, b), cu_seqlens


# speedup vs baseline: 1.2968x; 1.2968x over previous
import functools

import jax
import jax.numpy as jnp
from jax import lax
from jax.experimental import pallas as pl
from jax.experimental.pallas import tpu as pltpu
from jax.experimental.pallas import tpu_sc as plsc

TOTAL = 32768
D = 128
NC = 2
NS = 16
L = 16
NW = NC * NS
TPW = TOTAL // NW
CH = 256
NCH = TPW // CH
G = CH // L
NB = 2

_mesh = plsc.VectorSubcoreMesh(core_axis_name="c", subcore_axis_name="s")


@functools.partial(
    pl.kernel,
    mesh=_mesh,
    out_type=jax.ShapeDtypeStruct((TOTAL,), jnp.float32),
    compiler_params=pltpu.CompilerParams(needs_layout_passes=False),
    scratch_types=[
        pltpu.VMEM((TPW,), jnp.int32),
        pltpu.VMEM((NB, CH, D), jnp.float32),
        pltpu.VMEM((TPW,), jnp.float32),
        pltpu.VMEM((D, 1), jnp.float32),
        pltpu.VMEM((1,), jnp.float32),
        pltpu.SemaphoreType.DMA((NB,)),
    ],
)
def _sc_cost_kernel(tokens_hbm, table_hbm, w_hbm, b_hbm, out_hbm,
                    idx_v, rows, out_v, w_v, b_v, sems):
    wid = lax.axis_index("s") * NC + lax.axis_index("c")
    base = wid * TPW

    pltpu.sync_copy(w_hbm, w_v)
    pltpu.sync_copy(b_hbm, b_v)
    pltpu.sync_copy(tokens_hbm.at[pl.ds(base, TPW)], idx_v)

    def gather_copy(c):
        slot = lax.rem(c, NB)
        return pltpu.make_async_copy(
            table_hbm.at[idx_v.at[pl.ds(pl.multiple_of(c * CH, CH), CH)]],
            rows.at[slot], sems.at[slot])

    lane = lax.broadcasted_iota(jnp.int32, (L,), 0)
    zero16 = jnp.zeros((L,), jnp.int32)
    bias_vec = plsc.load_gather(b_v, [zero16])
    ridx = tuple(lane + g * L for g in range(G))

    for i in range(NB - 1):
        gather_copy(i).start()

    def chunk_body(c, _):
        @pl.when(c + NB - 1 < NCH)
        def _():
            gather_copy(c + NB - 1).start()

        slot = lax.rem(c, NB)
        gather_copy(c).wait()
        bsplat = zero16 + slot

        def body(i, accs):
            k = i & (L - 1)
            blk = i - k
            boff = ((lane + k) & (L - 1)) + blk
            wv = plsc.load_gather(w_v, [boff, zero16])
            return tuple(
                accs[g] + plsc.load_gather(rows, [bsplat, ridx[g], boff]) * wv
                for g in range(G))

        accs = lax.fori_loop(0, D, body, (bias_vec,) * G)
        for g in range(G):
            out_v[pl.ds(c * CH + g * L, L)] = accs[g]
        return 0

    lax.fori_loop(0, NCH, chunk_body, 0)
    pltpu.sync_copy(out_v, out_hbm.at[pl.ds(base, TPW)])


def kernel(tokens, cu_seqlens, table, W, b):
    return _sc_cost_kernel(tokens, table, W, b), cu_seqlens
